# Initial kernel scaffold; baseline (speedup 1.0000x reference)
import jax
import jax.numpy as jnp
from jax.experimental import pallas as pl


def kernel(x, edge_index, batch, params):
    raise NotImplementedError("write your pallas kernel here")


# speedup vs baseline: 4.1822x; 4.1822x over previous
import functools

import jax
import jax.numpy as jnp
from jax import lax
from jax.experimental import pallas as pl
from jax.experimental.pallas import tpu as pltpu
from jax.experimental.pallas import tpu_sc as plsc

_N = 10000
_E = 320000
_D = 128
_H = 128
_G = 64
_OUT = 128
_BN_EPS = 1e-5
_INV = (1.0 + _BN_EPS) ** -0.5

_NTILES = 32
_KCH = 128
_NCHUNK = 79
_EPT = _KCH * _NCHUNK
_EPAD = _EPT * _NTILES
_NPAD = 10240
_RPT = _NPAD // 16

_BR = 1000


def _make_segsum():
    mesh = plsc.VectorSubcoreMesh(core_axis_name="c", subcore_axis_name="s")

    @functools.partial(
        pl.kernel,
        out_type=(
            jax.ShapeDtypeStruct((_NPAD, _D), jnp.float32),
            jax.ShapeDtypeStruct((_NPAD, _D), jnp.float32),
        ),
        mesh=mesh,
        scratch_types=[
            pltpu.VMEM((_NCHUNK, _KCH), jnp.int32),
            pltpu.VMEM((_NCHUNK, _KCH), jnp.int32),
            pltpu.VMEM((_KCH, _D), jnp.float32),
            pltpu.VMEM_SHARED((_NPAD, _D), jnp.float32),
            pltpu.SemaphoreType.DMA,
        ],
    )
    def segsum(h_hbm, src_hbm, dst_hbm, out0_hbm, out1_hbm,
               sidx, didx, rows, aggsh, sem):
        c = lax.axis_index("c")
        s = lax.axis_index("s")
        wid = s * 2 + c

        pltpu.sync_copy(src_hbm.at[wid], sidx)
        pltpu.sync_copy(dst_hbm.at[wid], didx)

        zv = jnp.zeros((16,), jnp.float32)

        def _zrow(i, carry):
            for j in range(_D // 16):
                rows[i, pl.ds(j * 16, 16)] = zv
            return carry

        lax.fori_loop(0, _KCH, _zrow, 0)

        def _zcopy(k, carry):
            pltpu.sync_copy(rows, aggsh.at[pl.ds(s * _RPT + k * _KCH, _KCH)])
            return carry

        lax.fori_loop(0, _RPT // _KCH, _zcopy, 0)
        plsc.subcore_barrier()

        def _edge(k, carry):
            pltpu.async_copy(h_hbm.at[sidx.at[k]], rows, sem).wait()
            pltpu.sync_copy(rows, aggsh.at[didx.at[k]], add=True)
            return carry

        lax.fori_loop(0, _NCHUNK, _edge, 0)
        plsc.subcore_barrier()

        @pl.when(c == 0)
        def _():
            pltpu.sync_copy(aggsh.at[pl.ds(s * _RPT, _RPT)],
                            out0_hbm.at[pl.ds(s * _RPT, _RPT)])

        @pl.when(c == 1)
        def _():
            pltpu.sync_copy(aggsh.at[pl.ds(s * _RPT, _RPT)],
                            out1_hbm.at[pl.ds(s * _RPT, _RPT)])

    return segsum


_SEGSUM = _make_segsum()


def _mlp_body(eps_ref, h_ref, a0_ref, a1_ref, w1_ref, b1_ref, g1_ref,
              be1_ref, w2_ref, b2_ref, g2_ref, be2_ref, o_ref):
    scale = 1.0 + eps_ref[0, 0]
    z = h_ref[...] * scale + a0_ref[...] + a1_ref[...]
    y = jnp.dot(z, w1_ref[...], preferred_element_type=jnp.float32)
    y = (y + b1_ref[...]) * (_INV * g1_ref[...]) + be1_ref[...]
    y = jnp.where(y >= 0, y, 0.01 * y)
    y2 = jnp.dot(y, w2_ref[...], preferred_element_type=jnp.float32)
    y2 = (y2 + b2_ref[...]) * (_INV * g2_ref[...]) + be2_ref[...]
    o_ref[...] = jnp.where(y2 >= 0, y2, 0.01 * y2)


def _mlp(h, a0, a1, layer):
    grid = (_N // _BR,)
    row_spec = pl.BlockSpec((_BR, _D), lambda i: (i, 0))
    full = lambda shape: pl.BlockSpec(shape, lambda i: (0, 0))
    return pl.pallas_call(
        _mlp_body,
        grid=grid,
        in_specs=[
            pl.BlockSpec(memory_space=pltpu.SMEM),
            row_spec, row_spec, row_spec,
            full((_D, 2 * _H)), full((1, 2 * _H)), full((1, 2 * _H)),
            full((1, 2 * _H)),
            full((2 * _H, _H)), full((1, _H)), full((1, _H)), full((1, _H)),
        ],
        out_specs=pl.BlockSpec((_BR, _H), lambda i: (i, 0)),
        out_shape=jax.ShapeDtypeStruct((_N, _H), jnp.float32),
    )(
        layer["eps"].reshape(1, 1), h, a0, a1,
        layer["W1"], layer["b1"].reshape(1, -1), layer["g1"].reshape(1, -1),
        layer["be1"].reshape(1, -1),
        layer["W2"], layer["b2"].reshape(1, -1), layer["g2"].reshape(1, -1),
        layer["be2"].reshape(1, -1),
    )


def _pool_body(batch_ref, h_ref, w1_ref, b1_ref, g_ref, be_ref, w2_ref,
               b2_ref, o_ref):
    onehot = (batch_ref[...] ==
              lax.broadcasted_iota(jnp.int32, (_N, _G), 1)).astype(jnp.float32)
    pooled = lax.dot_general(onehot, h_ref[...], (((0,), (0,)), ((), ())),
                             preferred_element_type=jnp.float32)
    o = jnp.dot(pooled, w1_ref[...], preferred_element_type=jnp.float32)
    o = (o + b1_ref[...]) * (_INV * g_ref[...]) + be_ref[...]
    o = jnp.maximum(o, 0.0)
    o_ref[...] = jnp.dot(o, w2_ref[...],
                         preferred_element_type=jnp.float32) + b2_ref[...]


def _pool_head(batch2d, h, params):
    return pl.pallas_call(
        _pool_body,
        out_shape=jax.ShapeDtypeStruct((_G, _OUT), jnp.float32),
    )(
        batch2d, h,
        params["lin1_W"], params["lin1_b"].reshape(1, -1),
        params["bn1_g"].reshape(1, -1), params["bn1_b"].reshape(1, -1),
        params["lin2_W"], params["lin2_b"].reshape(1, -1),
    )


def kernel(x, edge_index, batch, params):
    pad = _EPAD - _E
    src = jnp.concatenate(
        [edge_index[0], jnp.zeros((pad,), jnp.int32)]).reshape(
            _NTILES, _NCHUNK, _KCH)
    dst = jnp.concatenate(
        [edge_index[1], jnp.full((pad,), _N, jnp.int32)]).reshape(
            _NTILES, _NCHUNK, _KCH)
    h = x
    for layer in params["layers"]:
        a0, a1 = _SEGSUM(h, src, dst)
        h = _mlp(h, a0, a1, layer)
    return _pool_head(batch.reshape(_N, 1), h, params)


# speedup vs baseline: 6.5666x; 1.5701x over previous
import functools

import jax
import jax.numpy as jnp
from jax import lax
from jax.experimental import pallas as pl
from jax.experimental.pallas import tpu as pltpu
from jax.experimental.pallas import tpu_sc as plsc

_N = 10000
_E = 320000
_D = 128
_H = 128
_G = 64
_OUT = 128
_BN_EPS = 1e-5
_INV = (1.0 + _BN_EPS) ** -0.5

_NTILES = 32
_KCH = 112
_NCHUNK = 90
_NBUF = 3
_NIDX = 6
_EPT = _KCH * _NCHUNK
_EPAD = _EPT * _NTILES
_NPAD = 10112
_RPT = _NPAD // 16

_BR = 1000


def _make_segsum():
    mesh = plsc.VectorSubcoreMesh(core_axis_name="c", subcore_axis_name="s")

    @functools.partial(
        pl.kernel,
        out_type=(
            jax.ShapeDtypeStruct((_NPAD, _D), jnp.float32),
            jax.ShapeDtypeStruct((_NPAD, _D), jnp.float32),
        ),
        mesh=mesh,
        scratch_types=[
            pltpu.VMEM((_NIDX, 2, _KCH), jnp.int32),
            pltpu.VMEM((_NBUF, _KCH, _D), jnp.float32),
            pltpu.VMEM_SHARED((_NPAD, _D), jnp.float32),
        ] + [pltpu.SemaphoreType.DMA] * (_NIDX + 2 * _NBUF),
    )
    def segsum(h_hbm, ei_hbm, out0_hbm, out1_hbm, idxr, rows, aggsh, *sems):
        isem = sems[:_NIDX]
        gsem = sems[_NIDX:_NIDX + _NBUF]
        ssem = sems[_NIDX + _NBUF:]
        c = lax.axis_index("c")
        s = lax.axis_index("s")
        wid = s * 2 + c

        for t in range(3):
            pltpu.async_copy(ei_hbm.at[wid, t], idxr.at[t], isem[t])

        zv = jnp.zeros((16,), jnp.float32)

        def _zrow(i, carry):
            for j in range(_D // 16):
                rows[0, i, pl.ds(j * 16, 16)] = zv
            return carry

        lax.fori_loop(0, _KCH, _zrow, 0)

        def _zcopy(k, carry):
            pltpu.sync_copy(rows.at[0, pl.ds(0, 79)],
                            aggsh.at[pl.ds(s * _RPT + k * 79, 79)])
            return carry

        lax.fori_loop(0, _RPT // 79, _zcopy, 0)

        pltpu.make_async_copy(ei_hbm.at[wid, 0], idxr.at[0], isem[0]).wait()
        pltpu.async_copy(h_hbm.at[idxr.at[0, 0]], rows.at[0], gsem[0])
        plsc.subcore_barrier()

        def _group(g, carry):
            for u in range(6):
                i = g * 6 + u
                s3, s13, s23 = u % 3, (u + 1) % 3, (u + 2) % 3
                s16, s36, s46 = (u + 1) % 6, (u + 3) % 6, (u + 4) % 6

                @pl.when(i >= 2)
                def _():
                    pltpu.make_async_copy(
                        rows.at[s13], aggsh.at[idxr.at[s46, 1]],
                        ssem[s13]).wait()

                @pl.when(i + 3 < _NCHUNK)
                def _():
                    pltpu.async_copy(ei_hbm.at[wid, i + 3], idxr.at[s36],
                                     isem[s36])

                @pl.when(i + 1 < _NCHUNK)
                def _():
                    pltpu.make_async_copy(ei_hbm.at[wid, 0], idxr.at[s16],
                                          isem[s16]).wait()
                    pltpu.async_copy(h_hbm.at[idxr.at[s16, 0]], rows.at[s13],
                                     gsem[s13])

                pltpu.make_async_copy(h_hbm.at[idxr.at[u, 0]], rows.at[s3],
                                      gsem[s3]).wait()
                pltpu.async_copy(rows.at[s3], aggsh.at[idxr.at[u, 1]],
                                 ssem[s3], add=True)
            return carry

        lax.fori_loop(0, _NCHUNK // 6, _group, 0)
        for k in range(_NCHUNK - 2, _NCHUNK):
            pltpu.make_async_copy(
                rows.at[k % 3], aggsh.at[idxr.at[k % 6, 1]],
                ssem[k % 3]).wait()
        plsc.subcore_barrier()

        @pl.when(c == 0)
        def _():
            pltpu.sync_copy(aggsh.at[pl.ds(s * _RPT, _RPT)],
                            out0_hbm.at[pl.ds(s * _RPT, _RPT)])

        @pl.when(c == 1)
        def _():
            pltpu.sync_copy(aggsh.at[pl.ds(s * _RPT, _RPT)],
                            out1_hbm.at[pl.ds(s * _RPT, _RPT)])

    return segsum


_SEGSUM = _make_segsum()


def _mlp_body(eps_ref, h_ref, a0_ref, a1_ref, w1_ref, b1_ref, g1_ref,
              be1_ref, w2_ref, b2_ref, g2_ref, be2_ref, o_ref):
    scale = 1.0 + eps_ref[0, 0]
    z = h_ref[...] * scale + a0_ref[...] + a1_ref[...]
    y = jnp.dot(z, w1_ref[...], preferred_element_type=jnp.float32)
    y = (y + b1_ref[...]) * (_INV * g1_ref[...]) + be1_ref[...]
    y = jnp.where(y >= 0, y, 0.01 * y)
    y2 = jnp.dot(y, w2_ref[...], preferred_element_type=jnp.float32)
    y2 = (y2 + b2_ref[...]) * (_INV * g2_ref[...]) + be2_ref[...]
    o_ref[...] = jnp.where(y2 >= 0, y2, 0.01 * y2)


def _mlp(h, a0, a1, layer):
    grid = (_N // _BR,)
    row_spec = pl.BlockSpec((_BR, _D), lambda i: (i, 0))
    full = lambda shape: pl.BlockSpec(shape, lambda i: (0, 0))
    return pl.pallas_call(
        _mlp_body,
        grid=grid,
        in_specs=[
            pl.BlockSpec(memory_space=pltpu.SMEM),
            row_spec, row_spec, row_spec,
            full((_D, 2 * _H)), full((1, 2 * _H)), full((1, 2 * _H)),
            full((1, 2 * _H)),
            full((2 * _H, _H)), full((1, _H)), full((1, _H)), full((1, _H)),
        ],
        out_specs=pl.BlockSpec((_BR, _H), lambda i: (i, 0)),
        out_shape=jax.ShapeDtypeStruct((_N, _H), jnp.float32),
    )(
        layer["eps"].reshape(1, 1), h, a0, a1,
        layer["W1"], layer["b1"].reshape(1, -1), layer["g1"].reshape(1, -1),
        layer["be1"].reshape(1, -1),
        layer["W2"], layer["b2"].reshape(1, -1), layer["g2"].reshape(1, -1),
        layer["be2"].reshape(1, -1),
    )


def _pool_body(batch_ref, h_ref, w1_ref, b1_ref, g_ref, be_ref, w2_ref,
               b2_ref, o_ref):
    onehot = (batch_ref[...] ==
              lax.broadcasted_iota(jnp.int32, (_N, _G), 1)).astype(jnp.float32)
    pooled = lax.dot_general(onehot, h_ref[...], (((0,), (0,)), ((), ())),
                             preferred_element_type=jnp.float32)
    o = jnp.dot(pooled, w1_ref[...], preferred_element_type=jnp.float32)
    o = (o + b1_ref[...]) * (_INV * g_ref[...]) + be_ref[...]
    o = jnp.maximum(o, 0.0)
    o_ref[...] = jnp.dot(o, w2_ref[...],
                         preferred_element_type=jnp.float32) + b2_ref[...]


def _pool_head(batch2d, h, params):
    return pl.pallas_call(
        _pool_body,
        out_shape=jax.ShapeDtypeStruct((_G, _OUT), jnp.float32),
    )(
        batch2d, h,
        params["lin1_W"], params["lin1_b"].reshape(1, -1),
        params["bn1_g"].reshape(1, -1), params["bn1_b"].reshape(1, -1),
        params["lin2_W"], params["lin2_b"].reshape(1, -1),
    )


def kernel(x, edge_index, batch, params):
    pad = _EPAD - _E
    src = jnp.concatenate(
        [edge_index[0], jnp.zeros((pad,), jnp.int32)]).reshape(
            _NTILES, _NCHUNK, _KCH)
    dst = jnp.concatenate(
        [edge_index[1], jnp.full((pad,), _N, jnp.int32)]).reshape(
            _NTILES, _NCHUNK, _KCH)
    ei = jnp.stack([src, dst], axis=2)
    h = x
    for layer in params["layers"]:
        a0, a1 = _SEGSUM(h, ei)
        h = _mlp(h, a0, a1, layer)
    return _pool_head(batch.reshape(_N, 1), h, params)


# speedup vs baseline: 6.5674x; 1.0001x over previous
import functools

import jax
import jax.numpy as jnp
from jax import lax
from jax.experimental import pallas as pl
from jax.experimental.pallas import tpu as pltpu
from jax.experimental.pallas import tpu_sc as plsc

_N = 10000
_E = 320000
_D = 128
_H = 128
_G = 64
_OUT = 128
_BN_EPS = 1e-5
_INV = (1.0 + _BN_EPS) ** -0.5

_NTILES = 32
_KCH = 112
_NCHUNK = 90
_NBUF = 3
_NIDX = 6
_EPT = _KCH * _NCHUNK
_EPAD = _EPT * _NTILES
_NPAD = 10112
_RPT = _NPAD // 16

_BR = 1000


def _make_segsum():
    mesh = plsc.VectorSubcoreMesh(core_axis_name="c", subcore_axis_name="s")

    @functools.partial(
        pl.kernel,
        out_type=(
            jax.ShapeDtypeStruct((_NPAD, _D), jnp.float32),
            jax.ShapeDtypeStruct((_NPAD, _D), jnp.float32),
        ),
        mesh=mesh,
        scratch_types=[
            pltpu.VMEM((_NIDX, 2, _KCH), jnp.int32),
            pltpu.VMEM((_NBUF, _KCH, _D), jnp.float32),
            pltpu.VMEM_SHARED((_NPAD, _D), jnp.float32),
        ] + [pltpu.SemaphoreType.DMA] * (_NIDX + 2 * _NBUF),
    )
    def segsum(h_hbm, ei_hbm, out0_hbm, out1_hbm, idxr, rows, aggsh, *sems):
        isem = sems[:_NIDX]
        gsem = sems[_NIDX:_NIDX + _NBUF]
        ssem = sems[_NIDX + _NBUF:]
        c = lax.axis_index("c")
        s = lax.axis_index("s")
        wid = s * 2 + c

        with jax.named_scope("sc_stage"):
            for t in range(3):
                pltpu.async_copy(ei_hbm.at[wid, t], idxr.at[t], isem[t])

        zv = jnp.zeros((16,), jnp.float32)

        with jax.named_scope("sc_zero"):
            def _zrow(i, carry):
                for j in range(_D // 16):
                    rows[0, i, pl.ds(j * 16, 16)] = zv
                return carry

            lax.fori_loop(0, _KCH, _zrow, 0)

            def _zcopy(k, carry):
                pltpu.sync_copy(rows.at[0, pl.ds(0, 79)],
                                aggsh.at[pl.ds(s * _RPT + k * 79, 79)])
                return carry

            lax.fori_loop(0, _RPT // 79, _zcopy, 0)

        pltpu.make_async_copy(ei_hbm.at[wid, 0], idxr.at[0], isem[0]).wait()
        pltpu.async_copy(h_hbm.at[idxr.at[0, 0]], rows.at[0], gsem[0])
        plsc.subcore_barrier()

        def _group(g, carry):
            for u in range(6):
                i = g * 6 + u
                s3, s13, s23 = u % 3, (u + 1) % 3, (u + 2) % 3
                s16, s36, s46 = (u + 1) % 6, (u + 3) % 6, (u + 4) % 6

                @pl.when(i >= 2)
                def _():
                    pltpu.make_async_copy(
                        rows.at[s13], aggsh.at[idxr.at[s46, 1]],
                        ssem[s13]).wait()

                @pl.when(i + 3 < _NCHUNK)
                def _():
                    pltpu.async_copy(ei_hbm.at[wid, i + 3], idxr.at[s36],
                                     isem[s36])

                @pl.when(i + 1 < _NCHUNK)
                def _():
                    pltpu.make_async_copy(ei_hbm.at[wid, 0], idxr.at[s16],
                                          isem[s16]).wait()
                    pltpu.async_copy(h_hbm.at[idxr.at[s16, 0]], rows.at[s13],
                                     gsem[s13])

                pltpu.make_async_copy(h_hbm.at[idxr.at[u, 0]], rows.at[s3],
                                      gsem[s3]).wait()
                pltpu.async_copy(rows.at[s3], aggsh.at[idxr.at[u, 1]],
                                 ssem[s3], add=True)
            return carry

        with jax.named_scope("sc_edges"):
            lax.fori_loop(0, _NCHUNK // 6, _group, 0)
            for k in range(_NCHUNK - 2, _NCHUNK):
                pltpu.make_async_copy(
                    rows.at[k % 3], aggsh.at[idxr.at[k % 6, 1]],
                    ssem[k % 3]).wait()
        plsc.subcore_barrier()

        with jax.named_scope("sc_flush"):
            @pl.when(c == 0)
            def _():
                pltpu.sync_copy(aggsh.at[pl.ds(s * _RPT, _RPT)],
                                out0_hbm.at[pl.ds(s * _RPT, _RPT)])

            @pl.when(c == 1)
            def _():
                pltpu.sync_copy(aggsh.at[pl.ds(s * _RPT, _RPT)],
                                out1_hbm.at[pl.ds(s * _RPT, _RPT)])

    return segsum


_SEGSUM = _make_segsum()


def _mlp_body(eps_ref, h_ref, a0_ref, a1_ref, w1_ref, b1_ref, g1_ref,
              be1_ref, w2_ref, b2_ref, g2_ref, be2_ref, o_ref):
    scale = 1.0 + eps_ref[0, 0]
    z = h_ref[...] * scale + a0_ref[...] + a1_ref[...]
    y = jnp.dot(z, w1_ref[...], preferred_element_type=jnp.float32)
    y = (y + b1_ref[...]) * (_INV * g1_ref[...]) + be1_ref[...]
    y = jnp.where(y >= 0, y, 0.01 * y)
    y2 = jnp.dot(y, w2_ref[...], preferred_element_type=jnp.float32)
    y2 = (y2 + b2_ref[...]) * (_INV * g2_ref[...]) + be2_ref[...]
    o_ref[...] = jnp.where(y2 >= 0, y2, 0.01 * y2)


def _mlp(h, a0, a1, layer):
    grid = (_N // _BR,)
    row_spec = pl.BlockSpec((_BR, _D), lambda i: (i, 0))
    full = lambda shape: pl.BlockSpec(shape, lambda i: (0, 0))
    return pl.pallas_call(
        _mlp_body,
        grid=grid,
        in_specs=[
            pl.BlockSpec(memory_space=pltpu.SMEM),
            row_spec, row_spec, row_spec,
            full((_D, 2 * _H)), full((1, 2 * _H)), full((1, 2 * _H)),
            full((1, 2 * _H)),
            full((2 * _H, _H)), full((1, _H)), full((1, _H)), full((1, _H)),
        ],
        out_specs=pl.BlockSpec((_BR, _H), lambda i: (i, 0)),
        out_shape=jax.ShapeDtypeStruct((_N, _H), jnp.float32),
    )(
        layer["eps"].reshape(1, 1), h, a0, a1,
        layer["W1"], layer["b1"].reshape(1, -1), layer["g1"].reshape(1, -1),
        layer["be1"].reshape(1, -1),
        layer["W2"], layer["b2"].reshape(1, -1), layer["g2"].reshape(1, -1),
        layer["be2"].reshape(1, -1),
    )


def _pool_body(batch_ref, h_ref, w1_ref, b1_ref, g_ref, be_ref, w2_ref,
               b2_ref, o_ref):
    onehot = (batch_ref[...] ==
              lax.broadcasted_iota(jnp.int32, (_N, _G), 1)).astype(jnp.float32)
    pooled = lax.dot_general(onehot, h_ref[...], (((0,), (0,)), ((), ())),
                             preferred_element_type=jnp.float32)
    o = jnp.dot(pooled, w1_ref[...], preferred_element_type=jnp.float32)
    o = (o + b1_ref[...]) * (_INV * g_ref[...]) + be_ref[...]
    o = jnp.maximum(o, 0.0)
    o_ref[...] = jnp.dot(o, w2_ref[...],
                         preferred_element_type=jnp.float32) + b2_ref[...]


def _pool_head(batch2d, h, params):
    return pl.pallas_call(
        _pool_body,
        out_shape=jax.ShapeDtypeStruct((_G, _OUT), jnp.float32),
    )(
        batch2d, h,
        params["lin1_W"], params["lin1_b"].reshape(1, -1),
        params["bn1_g"].reshape(1, -1), params["bn1_b"].reshape(1, -1),
        params["lin2_W"], params["lin2_b"].reshape(1, -1),
    )


def kernel(x, edge_index, batch, params):
    pad = _EPAD - _E
    src = jnp.concatenate(
        [edge_index[0], jnp.zeros((pad,), jnp.int32)]).reshape(
            _NTILES, _NCHUNK, _KCH)
    dst = jnp.concatenate(
        [edge_index[1], jnp.full((pad,), _N, jnp.int32)]).reshape(
            _NTILES, _NCHUNK, _KCH)
    ei = jnp.stack([src, dst], axis=2)
    h = x
    for layer in params["layers"]:
        a0, a1 = _SEGSUM(h, ei)
        h = _mlp(h, a0, a1, layer)
    return _pool_head(batch.reshape(_N, 1), h, params)


# speedup vs baseline: 11.9938x; 1.8262x over previous
import functools

import jax
import jax.numpy as jnp
from jax import lax
from jax.experimental import pallas as pl
from jax.experimental.pallas import tpu as pltpu
from jax.experimental.pallas import tpu_sc as plsc

_N = 10000
_E = 320000
_D = 128
_H = 128
_G = 64
_OUT = 128
_BN_EPS = 1e-5
_INV = (1.0 + _BN_EPS) ** -0.5

_NTILES = 32
_KCH = 112
_NCHUNK = 90
_NBUF = 3
_NIDX = 6
_EPT = _KCH * _NCHUNK
_EPAD = _EPT * _NTILES
_NPAD = 10112
_RPT = _NPAD // 16

_BR = 1000


def _make_segsum():
    mesh = plsc.VectorSubcoreMesh(core_axis_name="c", subcore_axis_name="s")

    @functools.partial(
        pl.kernel,
        out_type=(
            jax.ShapeDtypeStruct((_NPAD, _D), jnp.float32),
            jax.ShapeDtypeStruct((_NPAD, _D), jnp.float32),
        ),
        mesh=mesh,
        scratch_types=[
            pltpu.VMEM((_NIDX, 2, _KCH), jnp.int32),
            pltpu.VMEM((_NBUF, _KCH, _D), jnp.float32),
            pltpu.VMEM_SHARED((_NPAD, _D), jnp.float32),
        ] + [pltpu.SemaphoreType.DMA] * (_NIDX + 2 * _NBUF),
    )
    def segsum(h_hbm, ei_hbm, out0_hbm, out1_hbm, idxr, rows, aggsh, *sems):
        isem = sems[:_NIDX]
        gsem = sems[_NIDX:_NIDX + _NBUF]
        ssem = sems[_NIDX + _NBUF:]
        c = lax.axis_index("c")
        s = lax.axis_index("s")
        wid = s * 2 + c

        with jax.named_scope("sc_stage"):
            for t in range(3):
                pltpu.async_copy(ei_hbm.at[wid, t], idxr.at[t], isem[t])

        zv = jnp.zeros((16,), jnp.float32)

        with jax.named_scope("sc_zero"):
            def _zrow(i, carry):
                for j in range(_D // 16):
                    rows[0, i, pl.ds(j * 16, 16)] = zv
                return carry

            lax.fori_loop(0, _KCH, _zrow, 0)

            def _zcopy(k, carry):
                pltpu.sync_copy(rows.at[0, pl.ds(0, 79)],
                                aggsh.at[pl.ds(s * _RPT + k * 79, 79)])
                return carry

            lax.fori_loop(0, _RPT // 79, _zcopy, 0)

        pltpu.make_async_copy(ei_hbm.at[wid, 0], idxr.at[0], isem[0]).wait()
        pltpu.async_copy(h_hbm.at[idxr.at[0, 0]], rows.at[0], gsem[0])
        plsc.subcore_barrier()

        def _group(g, carry):
            for u in range(6):
                i = g * 6 + u
                s3, s13, s23 = u % 3, (u + 1) % 3, (u + 2) % 3
                s16, s36, s46 = (u + 1) % 6, (u + 3) % 6, (u + 4) % 6

                @pl.when(i >= 2)
                def _():
                    pltpu.make_async_copy(
                        rows.at[s13], aggsh.at[idxr.at[s46, 1]],
                        ssem[s13]).wait()

                @pl.when(i + 3 < _NCHUNK)
                def _():
                    pltpu.async_copy(ei_hbm.at[wid, i + 3], idxr.at[s36],
                                     isem[s36])

                @pl.when(i + 1 < _NCHUNK)
                def _():
                    pltpu.make_async_copy(ei_hbm.at[wid, 0], idxr.at[s16],
                                          isem[s16]).wait()
                    pltpu.async_copy(h_hbm.at[idxr.at[s16, 0]], rows.at[s13],
                                     gsem[s13])

                pltpu.make_async_copy(h_hbm.at[idxr.at[u, 0]], rows.at[s3],
                                      gsem[s3]).wait()
                pltpu.async_copy(rows.at[s3], aggsh.at[idxr.at[u, 1]],
                                 ssem[s3], add=True)
            return carry

        with jax.named_scope("sc_edges"):
            lax.fori_loop(0, _NCHUNK // 6, _group, 0)
            for k in range(_NCHUNK - 2, _NCHUNK):
                pltpu.make_async_copy(
                    rows.at[k % 3], aggsh.at[idxr.at[k % 6, 1]],
                    ssem[k % 3]).wait()
        plsc.subcore_barrier()

        with jax.named_scope("sc_flush"):
            @pl.when(c == 0)
            def _():
                pltpu.sync_copy(aggsh.at[pl.ds(s * _RPT, _RPT)],
                                out0_hbm.at[pl.ds(s * _RPT, _RPT)])

            @pl.when(c == 1)
            def _():
                pltpu.sync_copy(aggsh.at[pl.ds(s * _RPT, _RPT)],
                                out1_hbm.at[pl.ds(s * _RPT, _RPT)])

    return segsum


_SEGSUM = _make_segsum()


def _mlp_body(eps_ref, h_ref, a0_ref, a1_ref, w1_ref, b1_ref, g1_ref,
              be1_ref, w2_ref, b2_ref, g2_ref, be2_ref, o_ref):
    scale = 1.0 + eps_ref[0, 0]
    z = h_ref[...] * scale + a0_ref[...] + a1_ref[...]
    y = jnp.dot(z, w1_ref[...], preferred_element_type=jnp.float32)
    y = (y + b1_ref[...]) * (_INV * g1_ref[...]) + be1_ref[...]
    y = jnp.where(y >= 0, y, 0.01 * y)
    y2 = jnp.dot(y, w2_ref[...], preferred_element_type=jnp.float32)
    y2 = (y2 + b2_ref[...]) * (_INV * g2_ref[...]) + be2_ref[...]
    o_ref[...] = jnp.where(y2 >= 0, y2, 0.01 * y2)


def _mlp(h, a0, a1, layer):
    grid = (_N // _BR,)
    row_spec = pl.BlockSpec((_BR, _D), lambda i: (i, 0))
    full = lambda shape: pl.BlockSpec(shape, lambda i: (0, 0))
    return pl.pallas_call(
        _mlp_body,
        grid=grid,
        in_specs=[
            pl.BlockSpec(memory_space=pltpu.SMEM),
            row_spec, row_spec, row_spec,
            full((_D, 2 * _H)), full((1, 2 * _H)), full((1, 2 * _H)),
            full((1, 2 * _H)),
            full((2 * _H, _H)), full((1, _H)), full((1, _H)), full((1, _H)),
        ],
        out_specs=pl.BlockSpec((_BR, _H), lambda i: (i, 0)),
        out_shape=jax.ShapeDtypeStruct((_N, _H), jnp.float32),
    )(
        layer["eps"].reshape(1, 1), h, a0, a1,
        layer["W1"], layer["b1"].reshape(1, -1), layer["g1"].reshape(1, -1),
        layer["be1"].reshape(1, -1),
        layer["W2"], layer["b2"].reshape(1, -1), layer["g2"].reshape(1, -1),
        layer["be2"].reshape(1, -1),
    )


def _pool_body(batch_ref, h_ref, w1_ref, b1_ref, g_ref, be_ref, w2_ref,
               b2_ref, o_ref):
    onehot = (batch_ref[...] ==
              lax.broadcasted_iota(jnp.int32, (_N, _G), 1)).astype(jnp.float32)
    pooled = lax.dot_general(onehot, h_ref[...], (((0,), (0,)), ((), ())),
                             preferred_element_type=jnp.float32)
    o = jnp.dot(pooled, w1_ref[...], preferred_element_type=jnp.float32)
    o = (o + b1_ref[...]) * (_INV * g_ref[...]) + be_ref[...]
    o = jnp.maximum(o, 0.0)
    o_ref[...] = jnp.dot(o, w2_ref[...],
                         preferred_element_type=jnp.float32) + b2_ref[...]


def _pool_head(batch2d, h, params):
    return pl.pallas_call(
        _pool_body,
        out_shape=jax.ShapeDtypeStruct((_G, _OUT), jnp.float32),
    )(
        batch2d, h,
        params["lin1_W"], params["lin1_b"].reshape(1, -1),
        params["bn1_g"].reshape(1, -1), params["bn1_b"].reshape(1, -1),
        params["lin2_W"], params["lin2_b"].reshape(1, -1),
    )


def kernel(x, edge_index, batch, params):
    pad = _EPAD - _E
    iota = jnp.arange(pad, dtype=jnp.int32)
    src = jnp.concatenate(
        [edge_index[0], iota % _N]).reshape(_NTILES, _NCHUNK, _KCH)
    dst = jnp.concatenate(
        [edge_index[1], _N + iota % (_NPAD - _N)]).reshape(
            _NTILES, _NCHUNK, _KCH)
    ei = jnp.stack([src, dst], axis=2)
    h = x
    for layer in params["layers"]:
        a0, a1 = _SEGSUM(h, ei)
        h = _mlp(h, a0, a1, layer)
    return _pool_head(batch.reshape(_N, 1), h, params)


# speedup vs baseline: 12.7324x; 1.0616x over previous
import functools

import jax
import jax.numpy as jnp
import numpy as np
from jax import lax
from jax.experimental import pallas as pl
from jax.experimental.pallas import tpu as pltpu
from jax.experimental.pallas import tpu_sc as plsc

_N = 10000
_E = 320000
_D = 128
_H = 128
_G = 64
_OUT = 128
_BN_EPS = 1e-5
_INV = (1.0 + _BN_EPS) ** -0.5

_NTILES = 32
_KCH = 112
_NCHUNK = 90
_NBUF = 3
_NIDX = 6
_EPT = _KCH * _NCHUNK
_EPAD = _EPT * _NTILES
_NPAD = 10112
_RPT = _NPAD // 16

_BR = 2000

_PAD_N = _EPAD - _E
_PAD_BLOCK = jnp.asarray(np.stack([
    np.arange(_PAD_N, dtype=np.int32) % _N,
    _N + np.arange(_PAD_N, dtype=np.int32) % (_NPAD - _N),
]))


def _make_segsum():
    mesh = plsc.VectorSubcoreMesh(core_axis_name="c", subcore_axis_name="s")

    @functools.partial(
        pl.kernel,
        out_type=(
            jax.ShapeDtypeStruct((_NPAD, _D), jnp.float32),
            jax.ShapeDtypeStruct((_NPAD, _D), jnp.float32),
        ),
        mesh=mesh,
        scratch_types=[
            pltpu.VMEM((_NIDX, 2, _KCH), jnp.int32),
            pltpu.VMEM((_NBUF, _KCH, _D), jnp.float32),
            pltpu.VMEM_SHARED((_NPAD, _D), jnp.float32),
        ] + [pltpu.SemaphoreType.DMA] * (_NIDX + 2 * _NBUF),
    )
    def segsum(h_hbm, ei_hbm, out0_hbm, out1_hbm, idxr, rows, aggsh, *sems):
        isem = sems[:_NIDX]
        gsem = sems[_NIDX:_NIDX + _NBUF]
        ssem = sems[_NIDX + _NBUF:]
        c = lax.axis_index("c")
        s = lax.axis_index("s")
        wid = s * 2 + c

        with jax.named_scope("sc_stage"):
            for t in range(3):
                pltpu.async_copy(ei_hbm.at[wid, t], idxr.at[t], isem[t])

        zv = jnp.zeros((16,), jnp.float32)

        with jax.named_scope("sc_zero"):
            def _zrow(i, carry):
                for j in range(_D // 16):
                    rows[0, i, pl.ds(j * 16, 16)] = zv
                return carry

            lax.fori_loop(0, _KCH, _zrow, 0)

            def _zcopy(k, carry):
                pltpu.sync_copy(rows.at[0, pl.ds(0, 79)],
                                aggsh.at[pl.ds(s * _RPT + k * 79, 79)])
                return carry

            lax.fori_loop(0, _RPT // 79, _zcopy, 0)

        pltpu.make_async_copy(ei_hbm.at[wid, 0], idxr.at[0], isem[0]).wait()
        pltpu.async_copy(h_hbm.at[idxr.at[0, 0]], rows.at[0], gsem[0])
        plsc.subcore_barrier()

        def _group(g, carry):
            for u in range(6):
                i = g * 6 + u
                s3, s13, s23 = u % 3, (u + 1) % 3, (u + 2) % 3
                s16, s36, s46 = (u + 1) % 6, (u + 3) % 6, (u + 4) % 6

                @pl.when(i >= 2)
                def _():
                    pltpu.make_async_copy(
                        rows.at[s13], aggsh.at[idxr.at[s46, 1]],
                        ssem[s13]).wait()

                @pl.when(i + 3 < _NCHUNK)
                def _():
                    pltpu.async_copy(ei_hbm.at[wid, i + 3], idxr.at[s36],
                                     isem[s36])

                @pl.when(i + 1 < _NCHUNK)
                def _():
                    pltpu.make_async_copy(ei_hbm.at[wid, 0], idxr.at[s16],
                                          isem[s16]).wait()
                    pltpu.async_copy(h_hbm.at[idxr.at[s16, 0]], rows.at[s13],
                                     gsem[s13])

                pltpu.make_async_copy(h_hbm.at[idxr.at[u, 0]], rows.at[s3],
                                      gsem[s3]).wait()
                pltpu.async_copy(rows.at[s3], aggsh.at[idxr.at[u, 1]],
                                 ssem[s3], add=True)
            return carry

        with jax.named_scope("sc_edges"):
            lax.fori_loop(0, _NCHUNK // 6, _group, 0)
            for k in range(_NCHUNK - 2, _NCHUNK):
                pltpu.make_async_copy(
                    rows.at[k % 3], aggsh.at[idxr.at[k % 6, 1]],
                    ssem[k % 3]).wait()
        plsc.subcore_barrier()

        with jax.named_scope("sc_flush"):
            @pl.when(c == 0)
            def _():
                pltpu.sync_copy(aggsh.at[pl.ds(s * _RPT, _RPT)],
                                out0_hbm.at[pl.ds(s * _RPT, _RPT)])

            @pl.when(c == 1)
            def _():
                pltpu.sync_copy(aggsh.at[pl.ds(s * _RPT, _RPT)],
                                out1_hbm.at[pl.ds(s * _RPT, _RPT)])

    return segsum


_SEGSUM = _make_segsum()


def _mlp_body(eps_ref, h_ref, a0_ref, a1_ref, w1_ref, b1_ref, g1_ref,
              be1_ref, w2_ref, b2_ref, g2_ref, be2_ref, o_ref):
    scale = 1.0 + eps_ref[0, 0]
    z = h_ref[...] * scale + a0_ref[...] + a1_ref[...]
    y = jnp.dot(z, w1_ref[...], preferred_element_type=jnp.float32)
    y = (y + b1_ref[...]) * (_INV * g1_ref[...]) + be1_ref[...]
    y = jnp.where(y >= 0, y, 0.01 * y)
    y2 = jnp.dot(y, w2_ref[...], preferred_element_type=jnp.float32)
    y2 = (y2 + b2_ref[...]) * (_INV * g2_ref[...]) + be2_ref[...]
    o_ref[...] = jnp.where(y2 >= 0, y2, 0.01 * y2)


def _mlp(h, a0, a1, layer):
    grid = (_N // _BR,)
    row_spec = pl.BlockSpec((_BR, _D), lambda i: (i, 0))
    full = lambda shape: pl.BlockSpec(shape, lambda i: (0, 0))
    return pl.pallas_call(
        _mlp_body,
        grid=grid,
        in_specs=[
            pl.BlockSpec(memory_space=pltpu.SMEM),
            row_spec, row_spec, row_spec,
            full((_D, 2 * _H)), full((1, 2 * _H)), full((1, 2 * _H)),
            full((1, 2 * _H)),
            full((2 * _H, _H)), full((1, _H)), full((1, _H)), full((1, _H)),
        ],
        out_specs=pl.BlockSpec((_BR, _H), lambda i: (i, 0)),
        out_shape=jax.ShapeDtypeStruct((_N, _H), jnp.float32),
    )(
        layer["eps"].reshape(1, 1), h, a0, a1,
        layer["W1"], layer["b1"].reshape(1, -1), layer["g1"].reshape(1, -1),
        layer["be1"].reshape(1, -1),
        layer["W2"], layer["b2"].reshape(1, -1), layer["g2"].reshape(1, -1),
        layer["be2"].reshape(1, -1),
    )


def _pool_body(batch_ref, h_ref, w1_ref, b1_ref, g_ref, be_ref, w2_ref,
               b2_ref, o_ref):
    onehot = (batch_ref[...] ==
              lax.broadcasted_iota(jnp.int32, (_N, _G), 1)).astype(jnp.float32)
    pooled = lax.dot_general(onehot, h_ref[...], (((0,), (0,)), ((), ())),
                             preferred_element_type=jnp.float32)
    o = jnp.dot(pooled, w1_ref[...], preferred_element_type=jnp.float32)
    o = (o + b1_ref[...]) * (_INV * g_ref[...]) + be_ref[...]
    o = jnp.maximum(o, 0.0)
    o_ref[...] = jnp.dot(o, w2_ref[...],
                         preferred_element_type=jnp.float32) + b2_ref[...]


def _pool_head(batch2d, h, params):
    return pl.pallas_call(
        _pool_body,
        out_shape=jax.ShapeDtypeStruct((_G, _OUT), jnp.float32),
    )(
        batch2d, h,
        params["lin1_W"], params["lin1_b"].reshape(1, -1),
        params["bn1_g"].reshape(1, -1), params["bn1_b"].reshape(1, -1),
        params["lin2_W"], params["lin2_b"].reshape(1, -1),
    )


def kernel(x, edge_index, batch, params):
    ei = jnp.concatenate([edge_index, _PAD_BLOCK], axis=1)
    ei = ei.reshape(2, _NTILES, _NCHUNK, _KCH).transpose(1, 2, 0, 3)
    h = x
    for layer in params["layers"]:
        a0, a1 = _SEGSUM(h, ei)
        h = _mlp(h, a0, a1, layer)
    return _pool_head(batch.reshape(_N, 1), h, params)


# speedup vs baseline: 13.2449x; 1.0402x over previous
import functools

import jax
import jax.numpy as jnp
import numpy as np
from jax import lax
from jax.experimental import pallas as pl
from jax.experimental.pallas import tpu as pltpu
from jax.experimental.pallas import tpu_sc as plsc

_N = 10000
_E = 320000
_D = 128
_H = 128
_G = 64
_OUT = 128
_BN_EPS = 1e-5
_INV = (1.0 + _BN_EPS) ** -0.5

_NTILES = 32
_KCH = 80
_NCHUNK = 125
_NBUF = 3
_NIDX = 6
_EPT = _KCH * _NCHUNK
_NPAD = 10112
_RPT = _NPAD // 16

_BR = 2000


def _make_segsum():
    mesh = plsc.VectorSubcoreMesh(core_axis_name="c", subcore_axis_name="s")

    @functools.partial(
        pl.kernel,
        out_type=(
            jax.ShapeDtypeStruct((_NPAD, _D), jnp.float32),
            jax.ShapeDtypeStruct((_NPAD, _D), jnp.float32),
        ),
        mesh=mesh,
        scratch_types=[
            pltpu.VMEM((_NIDX, _KCH), jnp.int32),
            pltpu.VMEM((_NIDX, _KCH), jnp.int32),
            pltpu.VMEM((_NBUF, _KCH, _D), jnp.float32),
            pltpu.VMEM_SHARED((_NPAD, _D), jnp.float32),
        ] + [pltpu.SemaphoreType.DMA] * (_NIDX + 2 * _NBUF),
    )
    def segsum(h_hbm, ei_hbm, out0_hbm, out1_hbm, sidxr, didxr, rows, aggsh,
               *sems):
        isem = sems[:_NIDX]
        gsem = sems[_NIDX:_NIDX + _NBUF]
        ssem = sems[_NIDX + _NBUF:]
        c = lax.axis_index("c")
        s = lax.axis_index("s")
        wid = s * 2 + c
        base = wid * _EPT

        def _stage(t, slot):
            off = base + t * _KCH
            pltpu.async_copy(ei_hbm.at[pl.ds(off, _KCH)], sidxr.at[slot],
                             isem[slot])
            pltpu.async_copy(ei_hbm.at[pl.ds(_E + off, _KCH)],
                             didxr.at[slot], isem[slot])

        def _stage_wait(slot):
            pltpu.make_async_copy(ei_hbm.at[pl.ds(0, _KCH)],
                                  sidxr.at[slot], isem[slot]).wait()
            pltpu.make_async_copy(ei_hbm.at[pl.ds(0, _KCH)],
                                  didxr.at[slot], isem[slot]).wait()

        with jax.named_scope("sc_stage"):
            for t in range(3):
                _stage(t, t)

        zv = jnp.zeros((16,), jnp.float32)

        with jax.named_scope("sc_zero"):
            def _zrow(i, carry):
                for j in range(_D // 16):
                    rows[0, i, pl.ds(j * 16, 16)] = zv
                return carry

            lax.fori_loop(0, _KCH, _zrow, 0)

            def _zcopy(k, carry):
                pltpu.sync_copy(rows.at[0, pl.ds(0, 79)],
                                aggsh.at[pl.ds(s * _RPT + k * 79, 79)])
                return carry

            lax.fori_loop(0, _RPT // 79, _zcopy, 0)

        _stage_wait(0)
        pltpu.async_copy(h_hbm.at[sidxr.at[0]], rows.at[0], gsem[0])
        plsc.subcore_barrier()

        def _step(i, u, static_tail):
            s3, s13 = u % 3, (u + 1) % 3
            s16, s36, s46 = (u + 1) % 6, (u + 3) % 6, (u + 4) % 6

            def _retire():
                pltpu.make_async_copy(
                    rows.at[s13], aggsh.at[didxr.at[s46]], ssem[s13]).wait()

            def _prefetch():
                _stage(i + 3, s36)

            def _gather():
                _stage_wait(s16)
                pltpu.async_copy(h_hbm.at[sidxr.at[s16]], rows.at[s13],
                                 gsem[s13])

            if static_tail:
                _retire()
                if i + 3 < _NCHUNK:
                    _prefetch()
                if i + 1 < _NCHUNK:
                    _gather()
            else:
                pl.when(i >= 2)(_retire)
                pl.when(i + 3 < _NCHUNK)(_prefetch)
                pl.when(i + 1 < _NCHUNK)(_gather)

            pltpu.make_async_copy(h_hbm.at[sidxr.at[u]], rows.at[s3],
                                  gsem[s3]).wait()
            pltpu.async_copy(rows.at[s3], aggsh.at[didxr.at[u]],
                             ssem[s3], add=True)

        def _group(g, carry):
            for u in range(6):
                _step(g * 6 + u, u, False)
            return carry

        _tail = _NCHUNK % 6
        with jax.named_scope("sc_edges"):
            lax.fori_loop(0, _NCHUNK // 6, _group, 0)
            for i in range(_NCHUNK - _tail, _NCHUNK):
                _step(i, i % 6, True)
            for k in range(_NCHUNK - 2, _NCHUNK):
                pltpu.make_async_copy(
                    rows.at[k % 3], aggsh.at[didxr.at[k % 6]],
                    ssem[k % 3]).wait()
        plsc.subcore_barrier()

        with jax.named_scope("sc_flush"):
            @pl.when(c == 0)
            def _():
                pltpu.sync_copy(aggsh.at[pl.ds(s * _RPT, _RPT)],
                                out0_hbm.at[pl.ds(s * _RPT, _RPT)])

            @pl.when(c == 1)
            def _():
                pltpu.sync_copy(aggsh.at[pl.ds(s * _RPT, _RPT)],
                                out1_hbm.at[pl.ds(s * _RPT, _RPT)])

    return segsum


_SEGSUM = _make_segsum()


def _mlp_body(eps_ref, h_ref, a0_ref, a1_ref, w1_ref, b1_ref, g1_ref,
              be1_ref, w2_ref, b2_ref, g2_ref, be2_ref, o_ref):
    scale = 1.0 + eps_ref[0, 0]
    z = h_ref[...] * scale + a0_ref[...] + a1_ref[...]
    y = jnp.dot(z, w1_ref[...], preferred_element_type=jnp.float32)
    y = (y + b1_ref[...]) * (_INV * g1_ref[...]) + be1_ref[...]
    y = jnp.where(y >= 0, y, 0.01 * y)
    y2 = jnp.dot(y, w2_ref[...], preferred_element_type=jnp.float32)
    y2 = (y2 + b2_ref[...]) * (_INV * g2_ref[...]) + be2_ref[...]
    o_ref[...] = jnp.where(y2 >= 0, y2, 0.01 * y2)


def _mlp(h, a0, a1, layer):
    grid = (_N // _BR,)
    row_spec = pl.BlockSpec((_BR, _D), lambda i: (i, 0))
    full = lambda shape: pl.BlockSpec(shape, lambda i: (0, 0))
    return pl.pallas_call(
        _mlp_body,
        grid=grid,
        in_specs=[
            pl.BlockSpec(memory_space=pltpu.SMEM),
            row_spec, row_spec, row_spec,
            full((_D, 2 * _H)), full((1, 2 * _H)), full((1, 2 * _H)),
            full((1, 2 * _H)),
            full((2 * _H, _H)), full((1, _H)), full((1, _H)), full((1, _H)),
        ],
        out_specs=pl.BlockSpec((_BR, _H), lambda i: (i, 0)),
        out_shape=jax.ShapeDtypeStruct((_N, _H), jnp.float32),
    )(
        layer["eps"].reshape(1, 1), h, a0, a1,
        layer["W1"], layer["b1"].reshape(1, -1), layer["g1"].reshape(1, -1),
        layer["be1"].reshape(1, -1),
        layer["W2"], layer["b2"].reshape(1, -1), layer["g2"].reshape(1, -1),
        layer["be2"].reshape(1, -1),
    )


def _pool_body(batch_ref, h_ref, w1_ref, b1_ref, g_ref, be_ref, w2_ref,
               b2_ref, o_ref):
    onehot = (batch_ref[...] ==
              lax.broadcasted_iota(jnp.int32, (_N, _G), 1)).astype(jnp.float32)
    pooled = lax.dot_general(onehot, h_ref[...], (((0,), (0,)), ((), ())),
                             preferred_element_type=jnp.float32)
    o = jnp.dot(pooled, w1_ref[...], preferred_element_type=jnp.float32)
    o = (o + b1_ref[...]) * (_INV * g_ref[...]) + be_ref[...]
    o = jnp.maximum(o, 0.0)
    o_ref[...] = jnp.dot(o, w2_ref[...],
                         preferred_element_type=jnp.float32) + b2_ref[...]


def _pool_head(batch2d, h, params):
    return pl.pallas_call(
        _pool_body,
        out_shape=jax.ShapeDtypeStruct((_G, _OUT), jnp.float32),
    )(
        batch2d, h,
        params["lin1_W"], params["lin1_b"].reshape(1, -1),
        params["bn1_g"].reshape(1, -1), params["bn1_b"].reshape(1, -1),
        params["lin2_W"], params["lin2_b"].reshape(1, -1),
    )


def kernel(x, edge_index, batch, params):
    ei = edge_index.reshape(-1)
    h = x
    for layer in params["layers"]:
        a0, a1 = _SEGSUM(h, ei)
        h = _mlp(h, a0, a1, layer)
    return _pool_head(batch.reshape(_N, 1), h, params)
